# baseline (speedup 1.0000x reference)
import functools

import jax
import jax.numpy as jnp
from jax import lax
from jax.experimental import pallas as pl
from jax.experimental.pallas import tpu as pltpu
from jax.experimental.pallas import tpu_sc as plsc

N = 10000
E = 320000
P = 100000
NW = 32
EW = E // NW
CH = 128
NCH = 80
EPADW = NCH * CH
EPAD = NW * EPADW
NPAD = 10240
RT = NPAD // 16
ZB = 128
PCH = 128
PNCH = 26
PPAD = NW * PNCH * PCH

_mesh = plsc.VectorSubcoreMesh(core_axis_name="c", subcore_axis_name="s")


def _deg_sc(dst_r):
    @functools.partial(
        pl.kernel, mesh=_mesh,
        compiler_params=pltpu.CompilerParams(use_tc_tiling_on_sc=False),
        out_type=jax.ShapeDtypeStruct((2, NPAD, 128), jnp.float32),
        scratch_types=[
            pltpu.VMEM((NCH, CH), jnp.int32),
            pltpu.VMEM((CH, 16), jnp.float32),
            pltpu.VMEM((ZB, 16), jnp.float32),
            pltpu.VMEM_SHARED((NPAD, 16), jnp.float32),
        ],
    )
    def k(dstr_hbm, out_hbm, dstv, ones_v, zero_v, acc):
        c = lax.axis_index("c")
        s = lax.axis_index("s")
        wid = s * 2 + c
        pltpu.sync_copy(dstr_hbm.at[wid], dstv)

        def fill(i, _):
            ones_v[i] = jnp.full((16,), 1.0, jnp.float32)
            return 0
        lax.fori_loop(0, CH, fill, 0)

        def zfill(i, _):
            zero_v[i] = jnp.zeros((16,), jnp.float32)
            return 0
        lax.fori_loop(0, ZB, zfill, 0)
        for z in range(RT // ZB):
            pltpu.sync_copy(zero_v, acc.at[pl.ds(s * RT + z * ZB, ZB)])
        plsc.subcore_barrier()

        def chunk(j, _):
            pltpu.sync_copy(ones_v, acc.at[dstv.at[j]], add=True)
            return 0
        lax.fori_loop(0, NCH, chunk, 0)
        plsc.subcore_barrier()
        pltpu.sync_copy(acc.at[pl.ds(s * RT, RT)],
                        out_hbm.at[c, pl.ds(s * RT, RT), pl.ds(0, 16)])

    return k(dst_r)


def _scatter_sc(g, src_r, dst_r):
    D = g.shape[1]

    @functools.partial(
        pl.kernel, mesh=_mesh,
        compiler_params=pltpu.CompilerParams(use_tc_tiling_on_sc=False),
        out_type=jax.ShapeDtypeStruct((2, NPAD, 128), jnp.float32),
        scratch_types=[
            pltpu.VMEM((NCH, CH), jnp.int32),
            pltpu.VMEM((NCH, CH), jnp.int32),
            pltpu.VMEM((CH, D), jnp.float32),
            pltpu.VMEM((CH, D), jnp.float32),
            pltpu.VMEM((ZB, D), jnp.float32),
            pltpu.VMEM_SHARED((NPAD, D), jnp.float32),
            pltpu.SemaphoreType.DMA,
            pltpu.SemaphoreType.DMA,
        ],
    )
    def k(g_hbm, srcr_hbm, dstr_hbm, out_hbm, srcv, dstv, rows0, rows1,
          zero_v, acc, sem0, sem1):
        c = lax.axis_index("c")
        s = lax.axis_index("s")
        wid = s * 2 + c
        pltpu.sync_copy(srcr_hbm.at[wid], srcv)
        pltpu.sync_copy(dstr_hbm.at[wid], dstv)

        nsub = D // 16

        def zrow(t, _):
            zero_v[t // nsub, pl.ds((t % nsub) * 16, 16)] = jnp.zeros(
                (16,), jnp.float32)
            return 0
        lax.fori_loop(0, ZB * nsub, zrow, 0)
        for z in range(RT // ZB):
            pltpu.sync_copy(zero_v, acc.at[pl.ds(s * RT + z * ZB, ZB)])
        plsc.subcore_barrier()

        pltpu.async_copy(g_hbm.at[srcv.at[0]], rows0, sem0)

        def chunk2(jj, _):
            j0 = 2 * jj
            pltpu.make_async_copy(g_hbm.at[srcv.at[j0]], rows0, sem0).wait()
            pltpu.async_copy(g_hbm.at[srcv.at[j0 + 1]], rows1, sem1)
            pltpu.sync_copy(rows0, acc.at[dstv.at[j0]], add=True)
            pltpu.make_async_copy(g_hbm.at[srcv.at[j0 + 1]], rows1,
                                  sem1).wait()

            @pl.when(jj + 1 < NCH // 2)
            def _():
                pltpu.async_copy(g_hbm.at[srcv.at[j0 + 2]], rows0, sem0)

            pltpu.sync_copy(rows1, acc.at[dstv.at[j0 + 1]], add=True)
            return 0
        lax.fori_loop(0, NCH // 2, chunk2, 0)
        plsc.subcore_barrier()
        pltpu.sync_copy(acc.at[pl.ds(s * RT, RT)],
                        out_hbm.at[c, pl.ds(s * RT, RT), pl.ds(0, D)])

    return k(g, src_r, dst_r)


def _scatter2_sc(ga, gb, src_r2, dst_r2):
    NC2 = 2 * NCH

    @functools.partial(
        pl.kernel, mesh=_mesh,
        compiler_params=pltpu.CompilerParams(use_tc_tiling_on_sc=False),
        out_type=jax.ShapeDtypeStruct((2, NPAD, 128), jnp.float32),
        scratch_types=[
            pltpu.VMEM((NC2, CH), jnp.int32),
            pltpu.VMEM((NC2, CH), jnp.int32),
            pltpu.VMEM((CH, 64), jnp.float32),
            pltpu.VMEM((CH, 64), jnp.float32),
            pltpu.VMEM((ZB, 64), jnp.float32),
            pltpu.VMEM_SHARED((NPAD, 64), jnp.float32),
            pltpu.SemaphoreType.DMA,
            pltpu.SemaphoreType.DMA,
        ],
    )
    def k(ga_hbm, gb_hbm, srcr_hbm, dstr_hbm, out_hbm, srcv, dstv,
          rows0, rows1, zero_v, acc, sem0, sem1):
        c = lax.axis_index("c")
        s = lax.axis_index("s")
        pltpu.sync_copy(srcr_hbm.at[s], srcv)
        pltpu.sync_copy(dstr_hbm.at[s], dstv)

        def zrow(t, _):
            zero_v[t // 4, pl.ds((t % 4) * 16, 16)] = jnp.zeros(
                (16,), jnp.float32)
            return 0
        lax.fori_loop(0, ZB * 4, zrow, 0)
        for z in range(RT // ZB):
            pltpu.sync_copy(zero_v, acc.at[pl.ds(s * RT + z * ZB, ZB)])
        plsc.subcore_barrier()

        def run(tab):
            pltpu.async_copy(tab.at[srcv.at[0]], rows0, sem0)

            def chunk2(jj, _):
                j0 = 2 * jj
                pltpu.make_async_copy(tab.at[srcv.at[j0]], rows0,
                                      sem0).wait()
                pltpu.async_copy(tab.at[srcv.at[j0 + 1]], rows1, sem1)
                pltpu.sync_copy(rows0, acc.at[dstv.at[j0]], add=True)
                pltpu.make_async_copy(tab.at[srcv.at[j0 + 1]], rows1,
                                      sem1).wait()

                @pl.when(jj + 1 < NC2 // 2)
                def _():
                    pltpu.async_copy(tab.at[srcv.at[j0 + 2]], rows0, sem0)

                pltpu.sync_copy(rows1, acc.at[dstv.at[j0 + 1]], add=True)
                return 0
            lax.fori_loop(0, NC2 // 2, chunk2, 0)

        @pl.when(c == 0)
        def _():
            run(ga_hbm)

        @pl.when(c == 1)
        def _():
            run(gb_hbm)

        plsc.subcore_barrier()
        pltpu.sync_copy(acc.at[pl.ds(s * RT, RT)],
                        out_hbm.at[c, pl.ds(s * RT, RT), pl.ds(0, 64)])

    return k(ga, gb, src_r2, dst_r2)


def _pairgather_sc(ta, tb, u_r, v_r):
    @functools.partial(
        pl.kernel, mesh=_mesh,
        compiler_params=pltpu.CompilerParams(use_tc_tiling_on_sc=False),
        out_type=(jax.ShapeDtypeStruct((PPAD // PCH, PCH, 16), jnp.float32),
                  jax.ShapeDtypeStruct((PPAD // PCH, PCH, 16), jnp.float32)),
        scratch_types=[
            pltpu.VMEM((PNCH, PCH), jnp.int32),
            pltpu.VMEM((PNCH, PCH), jnp.int32),
            pltpu.VMEM((PCH, 16), jnp.float32),
            pltpu.VMEM((PCH, 16), jnp.float32),
            pltpu.VMEM((PCH, 16), jnp.float32),
            pltpu.VMEM((PCH, 16), jnp.float32),
            pltpu.SemaphoreType.DMA,
            pltpu.SemaphoreType.DMA,
            pltpu.SemaphoreType.DMA,
            pltpu.SemaphoreType.DMA,
        ],
    )
    def k(ta_hbm, tb_hbm, ur_hbm, vr_hbm, outa_hbm, outb_hbm, uv, vv,
          bufa0, bufb0, bufa1, bufb1, sa0, sb0, sa1, sb1):
        c = lax.axis_index("c")
        s = lax.axis_index("s")
        wid = s * 2 + c
        pltpu.sync_copy(ur_hbm.at[wid], uv)
        pltpu.sync_copy(vr_hbm.at[wid], vv)

        pltpu.async_copy(ta_hbm.at[uv.at[0]], bufa0, sa0)
        pltpu.async_copy(tb_hbm.at[vv.at[0]], bufb0, sb0)
        pltpu.async_copy(ta_hbm.at[uv.at[1]], bufa1, sa1)
        pltpu.async_copy(tb_hbm.at[vv.at[1]], bufb1, sb1)

        def chunk2(jj, _):
            j0 = 2 * jj
            for (j, ba, bb, sba, sbb) in ((j0, bufa0, bufb0, sa0, sb0),
                                          (j0 + 1, bufa1, bufb1, sa1, sb1)):
                row = wid * PNCH + j
                pltpu.make_async_copy(ta_hbm.at[uv.at[j]], ba, sba).wait()
                pltpu.make_async_copy(tb_hbm.at[vv.at[j]], bb, sbb).wait()
                pltpu.sync_copy(ba, outa_hbm.at[row])
                pltpu.sync_copy(bb, outb_hbm.at[row])

                @pl.when(j + 2 < PNCH)
                def _():
                    pltpu.async_copy(ta_hbm.at[uv.at[j + 2]], ba, sba)
                    pltpu.async_copy(tb_hbm.at[vv.at[j + 2]], bb, sbb)
            return 0
        lax.fori_loop(0, PNCH // 2, chunk2, 0)

    return k(ta, tb, u_r, v_r)


_BLK = 2000


def _tc_first(x, w1, degp):
    def body(x_ref, w_ref, d0_ref, d1_ref, ga_ref, gb_ref, dis_ref):
        deg = d0_ref[0, :, 0:1] + d1_ref[0, :, 0:1] + 1.0
        dis = lax.rsqrt(deg)
        h = jnp.dot(x_ref[...], w_ref[...], preferred_element_type=jnp.float32)
        g = dis * h
        ga_ref[...] = g[:, :64]
        gb_ref[...] = g[:, 64:]
        dis_ref[...] = jnp.broadcast_to(dis, (_BLK, 16))

    return pl.pallas_call(
        body,
        grid=(N // _BLK,),
        in_specs=[pl.BlockSpec((_BLK, 128), lambda i: (i, 0)),
                  pl.BlockSpec((128, 128), lambda i: (0, 0)),
                  pl.BlockSpec((1, _BLK, 128), lambda i: (0, i, 0)),
                  pl.BlockSpec((1, _BLK, 128), lambda i: (1, i, 0))],
        out_specs=[pl.BlockSpec((_BLK, 64), lambda i: (i, 0)),
                   pl.BlockSpec((_BLK, 64), lambda i: (i, 0)),
                   pl.BlockSpec((_BLK, 16), lambda i: (i, 0))],
        out_shape=[jax.ShapeDtypeStruct((N, 64), jnp.float32),
                   jax.ShapeDtypeStruct((N, 64), jnp.float32),
                   jax.ShapeDtypeStruct((N, 16), jnp.float32)],
    )(x, w1, degp, degp)


def _tc_mid2(pp, ga, gb, dis16, ba_row, bb_row, wa, wb):
    def body(pa_ref, pb_ref, ga_ref, gb_ref, dis_ref,
             ba_ref, bb_ref, wa_ref, wb_ref, out_ref):
        dis = dis_ref[:, 0:1]
        t_a = jnp.maximum(
            dis * (pa_ref[0, :, :64] + ga_ref[...]) + ba_ref[...], 0.0)
        t_b = jnp.maximum(
            dis * (pb_ref[0, :, :64] + gb_ref[...]) + bb_ref[...], 0.0)
        out_ref[...] = dis * (
            jnp.dot(t_a, wa_ref[...], preferred_element_type=jnp.float32)
            + jnp.dot(t_b, wb_ref[...], preferred_element_type=jnp.float32))

    blk64 = pl.BlockSpec((_BLK, 64), lambda i: (i, 0))
    pblk0 = pl.BlockSpec((1, _BLK, 128), lambda i: (0, i, 0))
    pblk1 = pl.BlockSpec((1, _BLK, 128), lambda i: (1, i, 0))
    return pl.pallas_call(
        body,
        grid=(N // _BLK,),
        in_specs=[pblk0, pblk1, blk64, blk64,
                  pl.BlockSpec((_BLK, 16), lambda i: (i, 0)),
                  pl.BlockSpec((1, 64), lambda i: (0, 0)),
                  pl.BlockSpec((1, 64), lambda i: (0, 0)),
                  pl.BlockSpec((64, 64), lambda i: (0, 0)),
                  pl.BlockSpec((64, 64), lambda i: (0, 0))],
        out_specs=pl.BlockSpec((_BLK, 64), lambda i: (i, 0)),
        out_shape=jax.ShapeDtypeStruct((N, 64), jnp.float32),
    )(pp, pp, ga, gb, dis16, ba_row, bb_row, wa, wb)


def _tc_mid(pp, g, dis16, b_row, w):
    din = g.shape[1]
    dout = w.shape[1]

    def body(p0_ref, p1_ref, g_ref, dis_ref, b_ref, w_ref, out_ref):
        dis = dis_ref[:, 0:1]
        p0 = p0_ref[0, :, :din]
        p1 = p1_ref[0, :, :din]
        t = dis * (p0 + p1 + g_ref[...]) + b_ref[...]
        t = jnp.maximum(t, 0.0)
        out_ref[...] = dis * jnp.dot(t, w_ref[...],
                                     preferred_element_type=jnp.float32)

    return pl.pallas_call(
        body,
        grid=(N // _BLK,),
        in_specs=[pl.BlockSpec((1, _BLK, 128), lambda i: (0, i, 0)),
                  pl.BlockSpec((1, _BLK, 128), lambda i: (1, i, 0)),
                  pl.BlockSpec((_BLK, din), lambda i: (i, 0)),
                  pl.BlockSpec((_BLK, 16), lambda i: (i, 0)),
                  pl.BlockSpec((1, din), lambda i: (0, 0)),
                  pl.BlockSpec((din, dout), lambda i: (0, 0))],
        out_specs=pl.BlockSpec((_BLK, dout), lambda i: (i, 0)),
        out_shape=jax.ShapeDtypeStruct((N, dout), jnp.float32),
    )(pp, pp, g, dis16, b_row, w)


def _tc_head(pp, g3, dis16, b3_row, l1, lb1_row, l2, lb2_row, m1, mb1_row):
    def body(p0_ref, p1_ref, g_ref, dis_ref, b3_ref, l1_ref, lb1_ref,
             l2_ref, lb2_ref, m1_ref, mb1_ref, ta_ref, tb_ref):
        dis = dis_ref[:, 0:1]
        o = dis * (p0_ref[0, :, :32] + p1_ref[0, :, :32] + g_ref[...]) \
            + b3_ref[...]
        o = jnp.maximum(o, 0.0)
        h4 = jnp.maximum(
            jnp.dot(o, l1_ref[...], preferred_element_type=jnp.float32)
            + lb1_ref[...], 0.0)
        emb = jnp.maximum(
            jnp.dot(h4, l2_ref[...], preferred_element_type=jnp.float32)
            + lb2_ref[...], 0.0)
        m1 = m1_ref[...]
        a = jnp.dot(emb, m1[:16, :], preferred_element_type=jnp.float32) \
            + mb1_ref[...]
        b = jnp.dot(emb, m1[16:, :], preferred_element_type=jnp.float32)
        ta_ref[...] = jnp.concatenate([a, a], axis=1)
        tb_ref[...] = jnp.concatenate([b, b], axis=1)

    return pl.pallas_call(
        body,
        grid=(N // _BLK,),
        in_specs=[pl.BlockSpec((1, _BLK, 128), lambda i: (0, i, 0)),
                  pl.BlockSpec((1, _BLK, 128), lambda i: (1, i, 0)),
                  pl.BlockSpec((_BLK, 32), lambda i: (i, 0)),
                  pl.BlockSpec((_BLK, 16), lambda i: (i, 0)),
                  pl.BlockSpec((1, 32), lambda i: (0, 0)),
                  pl.BlockSpec((32, 16), lambda i: (0, 0)),
                  pl.BlockSpec((1, 16), lambda i: (0, 0)),
                  pl.BlockSpec((16, 16), lambda i: (0, 0)),
                  pl.BlockSpec((1, 16), lambda i: (0, 0)),
                  pl.BlockSpec((32, 8), lambda i: (0, 0)),
                  pl.BlockSpec((1, 8), lambda i: (0, 0))],
        out_specs=[pl.BlockSpec((_BLK, 16), lambda i: (i, 0)),
                   pl.BlockSpec((_BLK, 16), lambda i: (i, 0))],
        out_shape=[jax.ShapeDtypeStruct((N, 16), jnp.float32),
                   jax.ShapeDtypeStruct((N, 16), jnp.float32)],
    )(pp, pp, g3, dis16, b3_row, l1, lb1_row, l2, lb2_row, m1, mb1_row)


def _tc_final(ga2, gb2, sel, mb2_s):
    rows = PPAD // 128
    blk = 104

    def body(a_ref, b_ref, sel_ref, mb2_ref, out_ref):
        t = jnp.maximum(a_ref[...] + b_ref[...], 0.0)
        sc = jnp.dot(t, sel_ref[...],
                     preferred_element_type=jnp.float32) + mb2_ref[...]
        out_ref[...] = 1.0 / (1.0 + jnp.exp(-sc))

    return pl.pallas_call(
        body,
        grid=(rows // blk,),
        in_specs=[pl.BlockSpec((blk, 2048), lambda i: (i, 0)),
                  pl.BlockSpec((blk, 2048), lambda i: (i, 0)),
                  pl.BlockSpec((2048, 128), lambda i: (0, 0)),
                  pl.BlockSpec((1, 1), lambda i: (0, 0))],
        out_specs=pl.BlockSpec((blk, 128), lambda i: (i, 0)),
        out_shape=jax.ShapeDtypeStruct((rows, 128), jnp.float32),
    )(ga2, gb2, sel, mb2_s)


def kernel(x, edge_index, pred_edges, W1, b1, W2, b2, W3, b3,
           L1, lb1, L2, lb2, M1, mb1, M2, mb2):
    ei = edge_index.astype(jnp.int32)
    npade = EPAD - E
    pad_src = jnp.arange(npade, dtype=jnp.int32) % N
    pad_dst = N + jnp.arange(npade, dtype=jnp.int32) % (NPAD - N)
    src_flat = jnp.concatenate([ei[0], pad_src])
    dst_flat = jnp.concatenate([ei[1], pad_dst])
    src_r = src_flat.reshape(NW, NCH, CH)
    dst_r = dst_flat.reshape(NW, NCH, CH)
    src_r2 = src_flat.reshape(16, 2 * NCH, CH)
    dst_r2 = dst_flat.reshape(16, 2 * NCH, CH)
    pe = pred_edges.astype(jnp.int32)
    npadp = PPAD - P
    pad_p = jnp.arange(npadp, dtype=jnp.int32) % N
    u_r = jnp.concatenate([pe[:, 0], pad_p]).reshape(NW, PNCH, PCH)
    v_r = jnp.concatenate([pe[:, 1], pad_p]).reshape(NW, PNCH, PCH)

    degp = _deg_sc(dst_r)
    g1a, g1b, dis16 = _tc_first(x, W1, degp)

    pp1 = _scatter2_sc(g1a, g1b, src_r2, dst_r2)
    g2 = _tc_mid2(pp1, g1a, g1b, dis16,
                  b1[:64].reshape(1, -1), b1[64:].reshape(1, -1),
                  W2[:64], W2[64:])

    pp = _scatter_sc(g2, src_r, dst_r)
    g3 = _tc_mid(pp, g2, dis16, b2.reshape(1, -1), W3)

    pp = _scatter_sc(g3, src_r, dst_r)
    ta, tb = _tc_head(pp, g3, dis16, b3.reshape(1, -1),
                      L1, lb1.reshape(1, -1), L2, lb2.reshape(1, -1),
                      M1, mb1.reshape(1, -1))

    ga, gb = _pairgather_sc(ta, tb, u_r, v_r)
    m2_pat = jnp.concatenate([M2[:, 0], jnp.zeros((8,), jnp.float32)])
    sel = jnp.kron(jnp.eye(128, dtype=jnp.float32), m2_pat.reshape(16, 1))
    y = _tc_final(ga.reshape(PPAD // PCH, PCH * 16),
                  gb.reshape(PPAD // PCH, PCH * 16),
                  sel, mb2.reshape(1, 1))
    return y.reshape(-1)[:P]


# speedup vs baseline: 1.1779x; 1.1779x over previous
import functools

import jax
import jax.numpy as jnp
from jax import lax
from jax.experimental import pallas as pl
from jax.experimental.pallas import tpu as pltpu
from jax.experimental.pallas import tpu_sc as plsc

N = 10000
E = 320000
P = 100000
NW = 32
EW = E // NW
CH = 128
NCH = 80
EPADW = NCH * CH
EPAD = NW * EPADW
NPAD = 10240
RT = NPAD // 16
ZB = 128
PCH = 128
PNCH = 26
PPAD = NW * PNCH * PCH

_mesh = plsc.VectorSubcoreMesh(core_axis_name="c", subcore_axis_name="s")


def _deg_sc(dst_r):
    @functools.partial(
        pl.kernel, mesh=_mesh,
        compiler_params=pltpu.CompilerParams(use_tc_tiling_on_sc=False),
        out_type=jax.ShapeDtypeStruct((2, NPAD, 128), jnp.float32),
        scratch_types=[
            pltpu.VMEM((NCH, CH), jnp.int32),
            pltpu.VMEM((CH, 16), jnp.float32),
            pltpu.VMEM((ZB, 16), jnp.float32),
            pltpu.VMEM_SHARED((NPAD, 16), jnp.float32),
        ],
    )
    def k(dstr_hbm, out_hbm, dstv, ones_v, zero_v, acc):
        c = lax.axis_index("c")
        s = lax.axis_index("s")
        wid = s * 2 + c
        pltpu.sync_copy(dstr_hbm.at[wid], dstv)

        def fill(i, _):
            ones_v[i] = jnp.full((16,), 1.0, jnp.float32)
            return 0
        lax.fori_loop(0, CH, fill, 0)

        def zfill(i, _):
            zero_v[i] = jnp.zeros((16,), jnp.float32)
            return 0
        lax.fori_loop(0, ZB, zfill, 0)
        for z in range(RT // ZB):
            pltpu.sync_copy(zero_v, acc.at[pl.ds(s * RT + z * ZB, ZB)])
        plsc.subcore_barrier()

        def chunk(j, _):
            pltpu.sync_copy(ones_v, acc.at[dstv.at[j]], add=True)
            return 0
        lax.fori_loop(0, NCH, chunk, 0)
        plsc.subcore_barrier()
        pltpu.sync_copy(acc.at[pl.ds(s * RT, RT)],
                        out_hbm.at[c, pl.ds(s * RT, RT), pl.ds(0, 16)])

    return k(dst_r)


def _scatter_sc(g, src_r, dst_r):
    D = g.shape[1]

    @functools.partial(
        pl.kernel, mesh=_mesh,
        compiler_params=pltpu.CompilerParams(use_tc_tiling_on_sc=False),
        out_type=jax.ShapeDtypeStruct((2, NPAD, 128), jnp.float32),
        scratch_types=[
            pltpu.VMEM((NCH, CH), jnp.int32),
            pltpu.VMEM((NCH, CH), jnp.int32),
            pltpu.VMEM((CH, D), jnp.float32),
            pltpu.VMEM((CH, D), jnp.float32),
            pltpu.VMEM((ZB, D), jnp.float32),
            pltpu.VMEM_SHARED((NPAD, D), jnp.float32),
            pltpu.SemaphoreType.DMA,
            pltpu.SemaphoreType.DMA,
        ],
    )
    def k(g_hbm, srcr_hbm, dstr_hbm, out_hbm, srcv, dstv, rows0, rows1,
          zero_v, acc, sem0, sem1):
        c = lax.axis_index("c")
        s = lax.axis_index("s")
        wid = s * 2 + c
        pltpu.sync_copy(srcr_hbm.at[wid], srcv)
        pltpu.sync_copy(dstr_hbm.at[wid], dstv)

        nsub = D // 16

        def zrow(t, _):
            zero_v[t // nsub, pl.ds((t % nsub) * 16, 16)] = jnp.zeros(
                (16,), jnp.float32)
            return 0
        lax.fori_loop(0, ZB * nsub, zrow, 0)
        for z in range(RT // ZB):
            pltpu.sync_copy(zero_v, acc.at[pl.ds(s * RT + z * ZB, ZB)])
        plsc.subcore_barrier()

        pltpu.async_copy(g_hbm.at[srcv.at[0]], rows0, sem0)

        def chunk2(jj, _):
            j0 = 2 * jj
            pltpu.make_async_copy(g_hbm.at[srcv.at[j0]], rows0, sem0).wait()
            pltpu.async_copy(g_hbm.at[srcv.at[j0 + 1]], rows1, sem1)
            pltpu.sync_copy(rows0, acc.at[dstv.at[j0]], add=True)
            pltpu.make_async_copy(g_hbm.at[srcv.at[j0 + 1]], rows1,
                                  sem1).wait()

            @pl.when(jj + 1 < NCH // 2)
            def _():
                pltpu.async_copy(g_hbm.at[srcv.at[j0 + 2]], rows0, sem0)

            pltpu.sync_copy(rows1, acc.at[dstv.at[j0 + 1]], add=True)
            return 0
        lax.fori_loop(0, NCH // 2, chunk2, 0)
        plsc.subcore_barrier()
        pltpu.sync_copy(acc.at[pl.ds(s * RT, RT)],
                        out_hbm.at[c, pl.ds(s * RT, RT), pl.ds(0, D)])

    return k(g, src_r, dst_r)


def _scatter2_sc(ga, gb, src_r2, dst_r2):
    NC2 = 2 * NCH

    @functools.partial(
        pl.kernel, mesh=_mesh,
        compiler_params=pltpu.CompilerParams(use_tc_tiling_on_sc=False),
        out_type=jax.ShapeDtypeStruct((2, NPAD, 128), jnp.float32),
        scratch_types=[
            pltpu.VMEM((NC2, CH), jnp.int32),
            pltpu.VMEM((NC2, CH), jnp.int32),
            pltpu.VMEM((CH, 64), jnp.float32),
            pltpu.VMEM((CH, 64), jnp.float32),
            pltpu.VMEM((ZB, 64), jnp.float32),
            pltpu.VMEM_SHARED((NPAD, 64), jnp.float32),
            pltpu.SemaphoreType.DMA,
            pltpu.SemaphoreType.DMA,
        ],
    )
    def k(ga_hbm, gb_hbm, srcr_hbm, dstr_hbm, out_hbm, srcv, dstv,
          rows0, rows1, zero_v, acc, sem0, sem1):
        c = lax.axis_index("c")
        s = lax.axis_index("s")
        pltpu.sync_copy(srcr_hbm.at[s], srcv)
        pltpu.sync_copy(dstr_hbm.at[s], dstv)

        def zrow(t, _):
            zero_v[t // 4, pl.ds((t % 4) * 16, 16)] = jnp.zeros(
                (16,), jnp.float32)
            return 0
        lax.fori_loop(0, ZB * 4, zrow, 0)
        for z in range(RT // ZB):
            pltpu.sync_copy(zero_v, acc.at[pl.ds(s * RT + z * ZB, ZB)])
        plsc.subcore_barrier()

        def run(tab):
            pltpu.async_copy(tab.at[srcv.at[0]], rows0, sem0)

            def chunk2(jj, _):
                j0 = 2 * jj
                pltpu.make_async_copy(tab.at[srcv.at[j0]], rows0,
                                      sem0).wait()
                pltpu.async_copy(tab.at[srcv.at[j0 + 1]], rows1, sem1)
                pltpu.sync_copy(rows0, acc.at[dstv.at[j0]], add=True)
                pltpu.make_async_copy(tab.at[srcv.at[j0 + 1]], rows1,
                                      sem1).wait()

                @pl.when(jj + 1 < NC2 // 2)
                def _():
                    pltpu.async_copy(tab.at[srcv.at[j0 + 2]], rows0, sem0)

                pltpu.sync_copy(rows1, acc.at[dstv.at[j0 + 1]], add=True)
                return 0
            lax.fori_loop(0, NC2 // 2, chunk2, 0)

        @pl.when(c == 0)
        def _():
            run(ga_hbm)

        @pl.when(c == 1)
        def _():
            run(gb_hbm)

        plsc.subcore_barrier()
        pltpu.sync_copy(acc.at[pl.ds(s * RT, RT)],
                        out_hbm.at[c, pl.ds(s * RT, RT), pl.ds(0, 64)])

    return k(ga, gb, src_r2, dst_r2)


def _pairgather_sc(ta, tb, u_r, v_r):
    @functools.partial(
        pl.kernel, mesh=_mesh,
        compiler_params=pltpu.CompilerParams(use_tc_tiling_on_sc=False),
        out_type=(jax.ShapeDtypeStruct((PPAD, 16), jnp.float32),
                  jax.ShapeDtypeStruct((PPAD, 16), jnp.float32)),
        scratch_types=[
            pltpu.VMEM((PNCH, PCH), jnp.int32),
            pltpu.VMEM((PNCH, PCH), jnp.int32),
            pltpu.VMEM((PCH, 16), jnp.float32),
            pltpu.VMEM((PCH, 16), jnp.float32),
            pltpu.VMEM((PCH, 16), jnp.float32),
            pltpu.VMEM((PCH, 16), jnp.float32),
            pltpu.SemaphoreType.DMA,
            pltpu.SemaphoreType.DMA,
            pltpu.SemaphoreType.DMA,
            pltpu.SemaphoreType.DMA,
        ],
    )
    def k(ta_hbm, tb_hbm, ur_hbm, vr_hbm, outa_hbm, outb_hbm, uv, vv,
          bufa0, bufb0, bufa1, bufb1, sa0, sb0, sa1, sb1):
        c = lax.axis_index("c")
        s = lax.axis_index("s")
        wid = s * 2 + c
        pltpu.sync_copy(ur_hbm.at[wid], uv)
        pltpu.sync_copy(vr_hbm.at[wid], vv)

        pltpu.async_copy(ta_hbm.at[uv.at[0]], bufa0, sa0)
        pltpu.async_copy(tb_hbm.at[vv.at[0]], bufb0, sb0)
        pltpu.async_copy(ta_hbm.at[uv.at[1]], bufa1, sa1)
        pltpu.async_copy(tb_hbm.at[vv.at[1]], bufb1, sb1)

        def chunk2(jj, _):
            j0 = 2 * jj
            for (j, ba, bb, sba, sbb) in ((j0, bufa0, bufb0, sa0, sb0),
                                          (j0 + 1, bufa1, bufb1, sa1, sb1)):
                base = (wid * PNCH + j) * PCH
                pltpu.make_async_copy(ta_hbm.at[uv.at[j]], ba, sba).wait()
                pltpu.make_async_copy(tb_hbm.at[vv.at[j]], bb, sbb).wait()
                pltpu.sync_copy(ba, outa_hbm.at[pl.ds(base, PCH)])
                pltpu.sync_copy(bb, outb_hbm.at[pl.ds(base, PCH)])

                @pl.when(j + 2 < PNCH)
                def _():
                    pltpu.async_copy(ta_hbm.at[uv.at[j + 2]], ba, sba)
                    pltpu.async_copy(tb_hbm.at[vv.at[j + 2]], bb, sbb)
            return 0
        lax.fori_loop(0, PNCH // 2, chunk2, 0)

    return k(ta, tb, u_r, v_r)


_BLK = 2000


def _tc_first(x, w1, degp):
    def body(x_ref, w_ref, d0_ref, d1_ref, ga_ref, gb_ref, dis_ref):
        deg = d0_ref[0, :, 0:1] + d1_ref[0, :, 0:1] + 1.0
        dis = lax.rsqrt(deg)
        h = jnp.dot(x_ref[...], w_ref[...], preferred_element_type=jnp.float32)
        g = dis * h
        ga_ref[...] = g[:, :64]
        gb_ref[...] = g[:, 64:]
        dis_ref[...] = jnp.broadcast_to(dis, (_BLK, 16))

    return pl.pallas_call(
        body,
        grid=(N // _BLK,),
        in_specs=[pl.BlockSpec((_BLK, 128), lambda i: (i, 0)),
                  pl.BlockSpec((128, 128), lambda i: (0, 0)),
                  pl.BlockSpec((1, _BLK, 128), lambda i: (0, i, 0)),
                  pl.BlockSpec((1, _BLK, 128), lambda i: (1, i, 0))],
        out_specs=[pl.BlockSpec((_BLK, 64), lambda i: (i, 0)),
                   pl.BlockSpec((_BLK, 64), lambda i: (i, 0)),
                   pl.BlockSpec((_BLK, 16), lambda i: (i, 0))],
        out_shape=[jax.ShapeDtypeStruct((N, 64), jnp.float32),
                   jax.ShapeDtypeStruct((N, 64), jnp.float32),
                   jax.ShapeDtypeStruct((N, 16), jnp.float32)],
    )(x, w1, degp, degp)


def _tc_mid2(pp, ga, gb, dis16, ba_row, bb_row, wa, wb):
    def body(pa_ref, pb_ref, ga_ref, gb_ref, dis_ref,
             ba_ref, bb_ref, wa_ref, wb_ref, out_ref):
        dis = dis_ref[:, 0:1]
        t_a = jnp.maximum(
            dis * (pa_ref[0, :, :64] + ga_ref[...]) + ba_ref[...], 0.0)
        t_b = jnp.maximum(
            dis * (pb_ref[0, :, :64] + gb_ref[...]) + bb_ref[...], 0.0)
        out_ref[...] = dis * (
            jnp.dot(t_a, wa_ref[...], preferred_element_type=jnp.float32)
            + jnp.dot(t_b, wb_ref[...], preferred_element_type=jnp.float32))

    blk64 = pl.BlockSpec((_BLK, 64), lambda i: (i, 0))
    pblk0 = pl.BlockSpec((1, _BLK, 128), lambda i: (0, i, 0))
    pblk1 = pl.BlockSpec((1, _BLK, 128), lambda i: (1, i, 0))
    return pl.pallas_call(
        body,
        grid=(N // _BLK,),
        in_specs=[pblk0, pblk1, blk64, blk64,
                  pl.BlockSpec((_BLK, 16), lambda i: (i, 0)),
                  pl.BlockSpec((1, 64), lambda i: (0, 0)),
                  pl.BlockSpec((1, 64), lambda i: (0, 0)),
                  pl.BlockSpec((64, 64), lambda i: (0, 0)),
                  pl.BlockSpec((64, 64), lambda i: (0, 0))],
        out_specs=pl.BlockSpec((_BLK, 64), lambda i: (i, 0)),
        out_shape=jax.ShapeDtypeStruct((N, 64), jnp.float32),
    )(pp, pp, ga, gb, dis16, ba_row, bb_row, wa, wb)


def _tc_mid(pp, g, dis16, b_row, w):
    din = g.shape[1]
    dout = w.shape[1]

    def body(p0_ref, p1_ref, g_ref, dis_ref, b_ref, w_ref, out_ref):
        dis = dis_ref[:, 0:1]
        p0 = p0_ref[0, :, :din]
        p1 = p1_ref[0, :, :din]
        t = dis * (p0 + p1 + g_ref[...]) + b_ref[...]
        t = jnp.maximum(t, 0.0)
        out_ref[...] = dis * jnp.dot(t, w_ref[...],
                                     preferred_element_type=jnp.float32)

    return pl.pallas_call(
        body,
        grid=(N // _BLK,),
        in_specs=[pl.BlockSpec((1, _BLK, 128), lambda i: (0, i, 0)),
                  pl.BlockSpec((1, _BLK, 128), lambda i: (1, i, 0)),
                  pl.BlockSpec((_BLK, din), lambda i: (i, 0)),
                  pl.BlockSpec((_BLK, 16), lambda i: (i, 0)),
                  pl.BlockSpec((1, din), lambda i: (0, 0)),
                  pl.BlockSpec((din, dout), lambda i: (0, 0))],
        out_specs=pl.BlockSpec((_BLK, dout), lambda i: (i, 0)),
        out_shape=jax.ShapeDtypeStruct((N, dout), jnp.float32),
    )(pp, pp, g, dis16, b_row, w)


def _tc_head(pp, g3, dis16, b3_row, l1, lb1_row, l2, lb2_row, m1, mb1_row):
    def body(p0_ref, p1_ref, g_ref, dis_ref, b3_ref, l1_ref, lb1_ref,
             l2_ref, lb2_ref, m1_ref, mb1_ref, ta_ref, tb_ref):
        dis = dis_ref[:, 0:1]
        o = dis * (p0_ref[0, :, :32] + p1_ref[0, :, :32] + g_ref[...]) \
            + b3_ref[...]
        o = jnp.maximum(o, 0.0)
        h4 = jnp.maximum(
            jnp.dot(o, l1_ref[...], preferred_element_type=jnp.float32)
            + lb1_ref[...], 0.0)
        emb = jnp.maximum(
            jnp.dot(h4, l2_ref[...], preferred_element_type=jnp.float32)
            + lb2_ref[...], 0.0)
        m1 = m1_ref[...]
        a = jnp.dot(emb, m1[:16, :], preferred_element_type=jnp.float32) \
            + mb1_ref[...]
        b = jnp.dot(emb, m1[16:, :], preferred_element_type=jnp.float32)
        ta_ref[...] = jnp.concatenate([a, a], axis=1)
        tb_ref[...] = jnp.concatenate([b, b], axis=1)

    return pl.pallas_call(
        body,
        grid=(N // _BLK,),
        in_specs=[pl.BlockSpec((1, _BLK, 128), lambda i: (0, i, 0)),
                  pl.BlockSpec((1, _BLK, 128), lambda i: (1, i, 0)),
                  pl.BlockSpec((_BLK, 32), lambda i: (i, 0)),
                  pl.BlockSpec((_BLK, 16), lambda i: (i, 0)),
                  pl.BlockSpec((1, 32), lambda i: (0, 0)),
                  pl.BlockSpec((32, 16), lambda i: (0, 0)),
                  pl.BlockSpec((1, 16), lambda i: (0, 0)),
                  pl.BlockSpec((16, 16), lambda i: (0, 0)),
                  pl.BlockSpec((1, 16), lambda i: (0, 0)),
                  pl.BlockSpec((32, 8), lambda i: (0, 0)),
                  pl.BlockSpec((1, 8), lambda i: (0, 0))],
        out_specs=[pl.BlockSpec((_BLK, 16), lambda i: (i, 0)),
                   pl.BlockSpec((_BLK, 16), lambda i: (i, 0))],
        out_shape=[jax.ShapeDtypeStruct((N, 16), jnp.float32),
                   jax.ShapeDtypeStruct((N, 16), jnp.float32)],
    )(pp, pp, g3, dis16, b3_row, l1, lb1_row, l2, lb2_row, m1, mb1_row)


def _tc_final(ga2, gb2, sel, mb2_s):
    rows = PPAD // 128
    blk = 104

    def body(a_ref, b_ref, sel_ref, mb2_ref, out_ref):
        t = jnp.maximum(a_ref[...] + b_ref[...], 0.0)
        sc = jnp.dot(t, sel_ref[...],
                     preferred_element_type=jnp.float32) + mb2_ref[...]
        out_ref[...] = 1.0 / (1.0 + jnp.exp(-sc))

    return pl.pallas_call(
        body,
        grid=(rows // blk,),
        in_specs=[pl.BlockSpec((blk, 2048), lambda i: (i, 0)),
                  pl.BlockSpec((blk, 2048), lambda i: (i, 0)),
                  pl.BlockSpec((2048, 128), lambda i: (0, 0)),
                  pl.BlockSpec((1, 1), lambda i: (0, 0))],
        out_specs=pl.BlockSpec((blk, 128), lambda i: (i, 0)),
        out_shape=jax.ShapeDtypeStruct((rows, 128), jnp.float32),
    )(ga2, gb2, sel, mb2_s)


def kernel(x, edge_index, pred_edges, W1, b1, W2, b2, W3, b3,
           L1, lb1, L2, lb2, M1, mb1, M2, mb2):
    ei = edge_index.astype(jnp.int32)
    npade = EPAD - E
    pad_src = jnp.arange(npade, dtype=jnp.int32) % N
    pad_dst = N + jnp.arange(npade, dtype=jnp.int32) % (NPAD - N)
    src_flat = jnp.concatenate([ei[0], pad_src])
    dst_flat = jnp.concatenate([ei[1], pad_dst])
    src_r = src_flat.reshape(NW, NCH, CH)
    dst_r = dst_flat.reshape(NW, NCH, CH)
    src_r2 = src_flat.reshape(16, 2 * NCH, CH)
    dst_r2 = dst_flat.reshape(16, 2 * NCH, CH)
    pe = pred_edges.astype(jnp.int32)
    npadp = PPAD - P
    pad_p = jnp.arange(npadp, dtype=jnp.int32) % N
    u_r = jnp.concatenate([pe[:, 0], pad_p]).reshape(NW, PNCH, PCH)
    v_r = jnp.concatenate([pe[:, 1], pad_p]).reshape(NW, PNCH, PCH)

    degp = _deg_sc(dst_r)
    g1a, g1b, dis16 = _tc_first(x, W1, degp)

    pp1 = _scatter2_sc(g1a, g1b, src_r2, dst_r2)
    g2 = _tc_mid2(pp1, g1a, g1b, dis16,
                  b1[:64].reshape(1, -1), b1[64:].reshape(1, -1),
                  W2[:64], W2[64:])

    pp = _scatter_sc(g2, src_r, dst_r)
    g3 = _tc_mid(pp, g2, dis16, b2.reshape(1, -1), W3)

    pp = _scatter_sc(g3, src_r, dst_r)
    ta, tb = _tc_head(pp, g3, dis16, b3.reshape(1, -1),
                      L1, lb1.reshape(1, -1), L2, lb2.reshape(1, -1),
                      M1, mb1.reshape(1, -1))

    ga, gb = _pairgather_sc(ta, tb, u_r, v_r)
    m2_pat = jnp.concatenate([M2[:, 0], jnp.zeros((8,), jnp.float32)])
    sel = jnp.kron(jnp.eye(128, dtype=jnp.float32), m2_pat.reshape(16, 1))
    y = _tc_final(ga.reshape(PPAD // PCH, PCH * 16),
                  gb.reshape(PPAD // PCH, PCH * 16),
                  sel, mb2.reshape(1, 1))
    return y.reshape(-1)[:P]


# speedup vs baseline: 1.3403x; 1.1378x over previous
import functools

import jax
import jax.numpy as jnp
from jax import lax
from jax.experimental import pallas as pl
from jax.experimental.pallas import tpu as pltpu
from jax.experimental.pallas import tpu_sc as plsc

N = 10000
E = 320000
P = 100000
NW = 32
EW = E // NW
CH = 128
NCH = 80
EPADW = NCH * CH
EPAD = NW * EPADW
NPAD = 10240
RT = NPAD // 16
ZB = 128
PCH = 128
PNCH = 26
PPAD = NW * PNCH * PCH

_mesh = plsc.VectorSubcoreMesh(core_axis_name="c", subcore_axis_name="s")


def _deg_sc(dst_r):
    @functools.partial(
        pl.kernel, mesh=_mesh,
        compiler_params=pltpu.CompilerParams(use_tc_tiling_on_sc=False),
        out_type=jax.ShapeDtypeStruct((2, NPAD, 128), jnp.float32),
        scratch_types=[
            pltpu.VMEM((NCH, CH), jnp.int32),
            pltpu.VMEM((CH, 16), jnp.float32),
            pltpu.VMEM((ZB, 16), jnp.float32),
            pltpu.VMEM_SHARED((NPAD, 16), jnp.float32),
        ],
    )
    def k(dstr_hbm, out_hbm, dstv, ones_v, zero_v, acc):
        c = lax.axis_index("c")
        s = lax.axis_index("s")
        wid = s * 2 + c
        pltpu.sync_copy(dstr_hbm.at[wid], dstv)

        def fill(i, _):
            ones_v[i] = jnp.full((16,), 1.0, jnp.float32)
            return 0
        lax.fori_loop(0, CH, fill, 0)

        def zfill(i, _):
            zero_v[i] = jnp.zeros((16,), jnp.float32)
            return 0
        lax.fori_loop(0, ZB, zfill, 0)
        for z in range(RT // ZB):
            pltpu.sync_copy(zero_v, acc.at[pl.ds(s * RT + z * ZB, ZB)])
        plsc.subcore_barrier()

        def chunk(j, _):
            pltpu.sync_copy(ones_v, acc.at[dstv.at[j]], add=True)
            return 0
        lax.fori_loop(0, NCH, chunk, 0)
        plsc.subcore_barrier()
        pltpu.sync_copy(acc.at[pl.ds(s * RT, RT)],
                        out_hbm.at[c, pl.ds(s * RT, RT), pl.ds(0, 16)])

    return k(dst_r)


def _scatter_sc(g, src_r, dst_r):
    D = g.shape[1]

    @functools.partial(
        pl.kernel, mesh=_mesh,
        compiler_params=pltpu.CompilerParams(use_tc_tiling_on_sc=False),
        out_type=jax.ShapeDtypeStruct((2, NPAD, 128), jnp.float32),
        scratch_types=[
            pltpu.VMEM((NCH, CH), jnp.int32),
            pltpu.VMEM((NCH, CH), jnp.int32),
            pltpu.VMEM((CH, D), jnp.float32),
            pltpu.VMEM((CH, D), jnp.float32),
            pltpu.VMEM((CH, D), jnp.float32),
            pltpu.VMEM((CH, D), jnp.float32),
            pltpu.VMEM((ZB, D), jnp.float32),
            pltpu.VMEM_SHARED((NPAD, D), jnp.float32),
            pltpu.SemaphoreType.DMA,
            pltpu.SemaphoreType.DMA,
            pltpu.SemaphoreType.DMA,
            pltpu.SemaphoreType.DMA,
        ],
    )
    def k(g_hbm, srcr_hbm, dstr_hbm, out_hbm, srcv, dstv, rows0, rows1,
          rows2, rows3, zero_v, acc, sem0, sem1, sem2, sem3):
        c = lax.axis_index("c")
        s = lax.axis_index("s")
        wid = s * 2 + c
        pltpu.sync_copy(srcr_hbm.at[wid], srcv)
        pltpu.sync_copy(dstr_hbm.at[wid], dstv)

        nsub = D // 16

        def zrow(t, _):
            zero_v[t // nsub, pl.ds((t % nsub) * 16, 16)] = jnp.zeros(
                (16,), jnp.float32)
            return 0
        lax.fori_loop(0, ZB * nsub, zrow, 0)
        for z in range(RT // ZB):
            pltpu.sync_copy(zero_v, acc.at[pl.ds(s * RT + z * ZB, ZB)])
        plsc.subcore_barrier()

        bufs = ((rows0, sem0), (rows1, sem1), (rows2, sem2), (rows3, sem3))
        pltpu.async_copy(g_hbm.at[srcv.at[0]], rows0, sem0)
        pltpu.async_copy(g_hbm.at[srcv.at[1]], rows1, sem1)

        def chunk4(jj, _):
            j0 = 4 * jj
            for b in range(4):
                rb, sb = bufs[b]
                rn, sn = bufs[(b + 2) % 4]
                pltpu.make_async_copy(g_hbm.at[srcv.at[j0 + b]], rb,
                                      sb).wait()

                @pl.when(j0 + b + 2 < NCH)
                def _():
                    pltpu.async_copy(g_hbm.at[srcv.at[j0 + b + 2]], rn, sn)

                pltpu.sync_copy(rb, acc.at[dstv.at[j0 + b]], add=True)
            return 0
        lax.fori_loop(0, NCH // 4, chunk4, 0)
        plsc.subcore_barrier()
        pltpu.sync_copy(acc.at[pl.ds(s * RT, RT)],
                        out_hbm.at[c, pl.ds(s * RT, RT), pl.ds(0, D)])

    return k(g, src_r, dst_r)


def _scatter2_sc(ga, gb, src_r2, dst_r2):
    NC2 = 2 * NCH

    @functools.partial(
        pl.kernel, mesh=_mesh,
        compiler_params=pltpu.CompilerParams(use_tc_tiling_on_sc=False),
        out_type=jax.ShapeDtypeStruct((2, NPAD, 128), jnp.float32),
        scratch_types=[
            pltpu.VMEM((NC2, CH), jnp.int32),
            pltpu.VMEM((NC2, CH), jnp.int32),
            pltpu.VMEM((CH, 64), jnp.float32),
            pltpu.VMEM((CH, 64), jnp.float32),
            pltpu.VMEM((ZB, 64), jnp.float32),
            pltpu.VMEM_SHARED((NPAD, 64), jnp.float32),
            pltpu.SemaphoreType.DMA,
            pltpu.SemaphoreType.DMA,
        ],
    )
    def k(ga_hbm, gb_hbm, srcr_hbm, dstr_hbm, out_hbm, srcv, dstv,
          rows0, rows1, zero_v, acc, sem0, sem1):
        c = lax.axis_index("c")
        s = lax.axis_index("s")
        pltpu.sync_copy(srcr_hbm.at[s], srcv)
        pltpu.sync_copy(dstr_hbm.at[s], dstv)

        def zrow(t, _):
            zero_v[t // 4, pl.ds((t % 4) * 16, 16)] = jnp.zeros(
                (16,), jnp.float32)
            return 0
        lax.fori_loop(0, ZB * 4, zrow, 0)
        for z in range(RT // ZB):
            pltpu.sync_copy(zero_v, acc.at[pl.ds(s * RT + z * ZB, ZB)])
        plsc.subcore_barrier()

        def run(tab):
            pltpu.async_copy(tab.at[srcv.at[0]], rows0, sem0)

            def chunk2(jj, _):
                j0 = 2 * jj
                pltpu.make_async_copy(tab.at[srcv.at[j0]], rows0,
                                      sem0).wait()
                pltpu.async_copy(tab.at[srcv.at[j0 + 1]], rows1, sem1)
                pltpu.sync_copy(rows0, acc.at[dstv.at[j0]], add=True)
                pltpu.make_async_copy(tab.at[srcv.at[j0 + 1]], rows1,
                                      sem1).wait()

                @pl.when(jj + 1 < NC2 // 2)
                def _():
                    pltpu.async_copy(tab.at[srcv.at[j0 + 2]], rows0, sem0)

                pltpu.sync_copy(rows1, acc.at[dstv.at[j0 + 1]], add=True)
                return 0
            lax.fori_loop(0, NC2 // 2, chunk2, 0)

        @pl.when(c == 0)
        def _():
            run(ga_hbm)

        @pl.when(c == 1)
        def _():
            run(gb_hbm)

        plsc.subcore_barrier()
        pltpu.sync_copy(acc.at[pl.ds(s * RT, RT)],
                        out_hbm.at[c, pl.ds(s * RT, RT), pl.ds(0, 64)])

    return k(ga, gb, src_r2, dst_r2)


def _pairgather_sc(ta, tb, u_r, v_r):
    @functools.partial(
        pl.kernel, mesh=_mesh,
        compiler_params=pltpu.CompilerParams(use_tc_tiling_on_sc=False),
        out_type=(jax.ShapeDtypeStruct((PPAD, 16), jnp.float32),
                  jax.ShapeDtypeStruct((PPAD, 16), jnp.float32)),
        scratch_types=[
            pltpu.VMEM((PNCH, PCH), jnp.int32),
            pltpu.VMEM((PNCH, PCH), jnp.int32),
            pltpu.VMEM((PCH, 16), jnp.float32),
            pltpu.VMEM((PCH, 16), jnp.float32),
            pltpu.VMEM((PCH, 16), jnp.float32),
            pltpu.VMEM((PCH, 16), jnp.float32),
            pltpu.SemaphoreType.DMA,
            pltpu.SemaphoreType.DMA,
            pltpu.SemaphoreType.DMA,
            pltpu.SemaphoreType.DMA,
        ],
    )
    def k(ta_hbm, tb_hbm, ur_hbm, vr_hbm, outa_hbm, outb_hbm, uv, vv,
          bufa0, bufb0, bufa1, bufb1, sa0, sb0, sa1, sb1):
        c = lax.axis_index("c")
        s = lax.axis_index("s")
        wid = s * 2 + c
        pltpu.sync_copy(ur_hbm.at[wid], uv)
        pltpu.sync_copy(vr_hbm.at[wid], vv)

        pltpu.async_copy(ta_hbm.at[uv.at[0]], bufa0, sa0)
        pltpu.async_copy(tb_hbm.at[vv.at[0]], bufb0, sb0)
        pltpu.async_copy(ta_hbm.at[uv.at[1]], bufa1, sa1)
        pltpu.async_copy(tb_hbm.at[vv.at[1]], bufb1, sb1)

        def chunk2(jj, _):
            j0 = 2 * jj
            for (j, ba, bb, sba, sbb) in ((j0, bufa0, bufb0, sa0, sb0),
                                          (j0 + 1, bufa1, bufb1, sa1, sb1)):
                base = (wid * PNCH + j) * PCH
                pltpu.make_async_copy(ta_hbm.at[uv.at[j]], ba, sba).wait()
                pltpu.make_async_copy(tb_hbm.at[vv.at[j]], bb, sbb).wait()
                pltpu.sync_copy(ba, outa_hbm.at[pl.ds(base, PCH)])
                pltpu.sync_copy(bb, outb_hbm.at[pl.ds(base, PCH)])

                @pl.when(j + 2 < PNCH)
                def _():
                    pltpu.async_copy(ta_hbm.at[uv.at[j + 2]], ba, sba)
                    pltpu.async_copy(tb_hbm.at[vv.at[j + 2]], bb, sbb)
            return 0
        lax.fori_loop(0, PNCH // 2, chunk2, 0)

    return k(ta, tb, u_r, v_r)


_BLK = 2000


def _tc_first(x, w1, degp):
    def body(x_ref, w_ref, d0_ref, d1_ref, ga_ref, gb_ref, dis_ref):
        deg = d0_ref[0, :, 0:1] + d1_ref[0, :, 0:1] + 1.0
        dis = lax.rsqrt(deg)
        h = jnp.dot(x_ref[...], w_ref[...], preferred_element_type=jnp.float32)
        g = dis * h
        ga_ref[...] = g[:, :64]
        gb_ref[...] = g[:, 64:]
        dis_ref[...] = jnp.broadcast_to(dis, (_BLK, 16))

    return pl.pallas_call(
        body,
        grid=(N // _BLK,),
        in_specs=[pl.BlockSpec((_BLK, 128), lambda i: (i, 0)),
                  pl.BlockSpec((128, 128), lambda i: (0, 0)),
                  pl.BlockSpec((1, _BLK, 128), lambda i: (0, i, 0)),
                  pl.BlockSpec((1, _BLK, 128), lambda i: (1, i, 0))],
        out_specs=[pl.BlockSpec((_BLK, 64), lambda i: (i, 0)),
                   pl.BlockSpec((_BLK, 64), lambda i: (i, 0)),
                   pl.BlockSpec((_BLK, 16), lambda i: (i, 0))],
        out_shape=[jax.ShapeDtypeStruct((N, 64), jnp.float32),
                   jax.ShapeDtypeStruct((N, 64), jnp.float32),
                   jax.ShapeDtypeStruct((N, 16), jnp.float32)],
    )(x, w1, degp, degp)


def _tc_mid2(pp, ga, gb, dis16, ba_row, bb_row, wa, wb):
    def body(pa_ref, pb_ref, ga_ref, gb_ref, dis_ref,
             ba_ref, bb_ref, wa_ref, wb_ref, out_ref):
        dis = dis_ref[:, 0:1]
        t_a = jnp.maximum(
            dis * (pa_ref[0, :, :64] + ga_ref[...]) + ba_ref[...], 0.0)
        t_b = jnp.maximum(
            dis * (pb_ref[0, :, :64] + gb_ref[...]) + bb_ref[...], 0.0)
        out_ref[...] = dis * (
            jnp.dot(t_a, wa_ref[...], preferred_element_type=jnp.float32)
            + jnp.dot(t_b, wb_ref[...], preferred_element_type=jnp.float32))

    blk64 = pl.BlockSpec((_BLK, 64), lambda i: (i, 0))
    pblk0 = pl.BlockSpec((1, _BLK, 128), lambda i: (0, i, 0))
    pblk1 = pl.BlockSpec((1, _BLK, 128), lambda i: (1, i, 0))
    return pl.pallas_call(
        body,
        grid=(N // _BLK,),
        in_specs=[pblk0, pblk1, blk64, blk64,
                  pl.BlockSpec((_BLK, 16), lambda i: (i, 0)),
                  pl.BlockSpec((1, 64), lambda i: (0, 0)),
                  pl.BlockSpec((1, 64), lambda i: (0, 0)),
                  pl.BlockSpec((64, 64), lambda i: (0, 0)),
                  pl.BlockSpec((64, 64), lambda i: (0, 0))],
        out_specs=pl.BlockSpec((_BLK, 64), lambda i: (i, 0)),
        out_shape=jax.ShapeDtypeStruct((N, 64), jnp.float32),
    )(pp, pp, ga, gb, dis16, ba_row, bb_row, wa, wb)


def _tc_mid(pp, g, dis16, b_row, w):
    din = g.shape[1]
    dout = w.shape[1]

    def body(p0_ref, p1_ref, g_ref, dis_ref, b_ref, w_ref, out_ref):
        dis = dis_ref[:, 0:1]
        p0 = p0_ref[0, :, :din]
        p1 = p1_ref[0, :, :din]
        t = dis * (p0 + p1 + g_ref[...]) + b_ref[...]
        t = jnp.maximum(t, 0.0)
        out_ref[...] = dis * jnp.dot(t, w_ref[...],
                                     preferred_element_type=jnp.float32)

    return pl.pallas_call(
        body,
        grid=(N // _BLK,),
        in_specs=[pl.BlockSpec((1, _BLK, 128), lambda i: (0, i, 0)),
                  pl.BlockSpec((1, _BLK, 128), lambda i: (1, i, 0)),
                  pl.BlockSpec((_BLK, din), lambda i: (i, 0)),
                  pl.BlockSpec((_BLK, 16), lambda i: (i, 0)),
                  pl.BlockSpec((1, din), lambda i: (0, 0)),
                  pl.BlockSpec((din, dout), lambda i: (0, 0))],
        out_specs=pl.BlockSpec((_BLK, dout), lambda i: (i, 0)),
        out_shape=jax.ShapeDtypeStruct((N, dout), jnp.float32),
    )(pp, pp, g, dis16, b_row, w)


def _tc_head(pp, g3, dis16, b3_row, l1, lb1_row, l2, lb2_row, m1, mb1_row):
    def body(p0_ref, p1_ref, g_ref, dis_ref, b3_ref, l1_ref, lb1_ref,
             l2_ref, lb2_ref, m1_ref, mb1_ref, ta_ref, tb_ref):
        dis = dis_ref[:, 0:1]
        o = dis * (p0_ref[0, :, :32] + p1_ref[0, :, :32] + g_ref[...]) \
            + b3_ref[...]
        o = jnp.maximum(o, 0.0)
        h4 = jnp.maximum(
            jnp.dot(o, l1_ref[...], preferred_element_type=jnp.float32)
            + lb1_ref[...], 0.0)
        emb = jnp.maximum(
            jnp.dot(h4, l2_ref[...], preferred_element_type=jnp.float32)
            + lb2_ref[...], 0.0)
        m1 = m1_ref[...]
        a = jnp.dot(emb, m1[:16, :], preferred_element_type=jnp.float32) \
            + mb1_ref[...]
        b = jnp.dot(emb, m1[16:, :], preferred_element_type=jnp.float32)
        ta_ref[...] = jnp.concatenate([a, a], axis=1)
        tb_ref[...] = jnp.concatenate([b, b], axis=1)

    return pl.pallas_call(
        body,
        grid=(N // _BLK,),
        in_specs=[pl.BlockSpec((1, _BLK, 128), lambda i: (0, i, 0)),
                  pl.BlockSpec((1, _BLK, 128), lambda i: (1, i, 0)),
                  pl.BlockSpec((_BLK, 32), lambda i: (i, 0)),
                  pl.BlockSpec((_BLK, 16), lambda i: (i, 0)),
                  pl.BlockSpec((1, 32), lambda i: (0, 0)),
                  pl.BlockSpec((32, 16), lambda i: (0, 0)),
                  pl.BlockSpec((1, 16), lambda i: (0, 0)),
                  pl.BlockSpec((16, 16), lambda i: (0, 0)),
                  pl.BlockSpec((1, 16), lambda i: (0, 0)),
                  pl.BlockSpec((32, 8), lambda i: (0, 0)),
                  pl.BlockSpec((1, 8), lambda i: (0, 0))],
        out_specs=[pl.BlockSpec((_BLK, 16), lambda i: (i, 0)),
                   pl.BlockSpec((_BLK, 16), lambda i: (i, 0))],
        out_shape=[jax.ShapeDtypeStruct((N, 16), jnp.float32),
                   jax.ShapeDtypeStruct((N, 16), jnp.float32)],
    )(pp, pp, g3, dis16, b3_row, l1, lb1_row, l2, lb2_row, m1, mb1_row)


def _tc_final(ga2, gb2, sel, mb2_s):
    rows = PPAD // 128
    blk = 104

    def body(a_ref, b_ref, sel_ref, mb2_ref, out_ref):
        t = jnp.maximum(a_ref[...] + b_ref[...], 0.0)
        sc = jnp.dot(t, sel_ref[...],
                     preferred_element_type=jnp.float32) + mb2_ref[...]
        out_ref[...] = 1.0 / (1.0 + jnp.exp(-sc))

    return pl.pallas_call(
        body,
        grid=(rows // blk,),
        in_specs=[pl.BlockSpec((blk, 2048), lambda i: (i, 0)),
                  pl.BlockSpec((blk, 2048), lambda i: (i, 0)),
                  pl.BlockSpec((2048, 128), lambda i: (0, 0)),
                  pl.BlockSpec((1, 1), lambda i: (0, 0))],
        out_specs=pl.BlockSpec((blk, 128), lambda i: (i, 0)),
        out_shape=jax.ShapeDtypeStruct((rows, 128), jnp.float32),
    )(ga2, gb2, sel, mb2_s)


def kernel(x, edge_index, pred_edges, W1, b1, W2, b2, W3, b3,
           L1, lb1, L2, lb2, M1, mb1, M2, mb2):
    ei = edge_index.astype(jnp.int32)
    npade = EPAD - E
    pad_src = jnp.arange(npade, dtype=jnp.int32) % N
    pad_dst = N + jnp.arange(npade, dtype=jnp.int32) % (NPAD - N)
    src_flat = jnp.concatenate([ei[0], pad_src])
    dst_flat = jnp.concatenate([ei[1], pad_dst])
    src_r = src_flat.reshape(NW, NCH, CH)
    dst_r = dst_flat.reshape(NW, NCH, CH)
    src_r2 = src_flat.reshape(16, 2 * NCH, CH)
    dst_r2 = dst_flat.reshape(16, 2 * NCH, CH)
    pe = pred_edges.astype(jnp.int32)
    npadp = PPAD - P
    pad_p = jnp.arange(npadp, dtype=jnp.int32) % N
    u_r = jnp.concatenate([pe[:, 0], pad_p]).reshape(NW, PNCH, PCH)
    v_r = jnp.concatenate([pe[:, 1], pad_p]).reshape(NW, PNCH, PCH)

    degp = _deg_sc(dst_r)
    g1a, g1b, dis16 = _tc_first(x, W1, degp)

    pp1 = _scatter2_sc(g1a, g1b, src_r2, dst_r2)
    g2 = _tc_mid2(pp1, g1a, g1b, dis16,
                  b1[:64].reshape(1, -1), b1[64:].reshape(1, -1),
                  W2[:64], W2[64:])

    pp = _scatter_sc(g2, src_r, dst_r)
    g3 = _tc_mid(pp, g2, dis16, b2.reshape(1, -1), W3)

    pp = _scatter_sc(g3, src_r, dst_r)
    ta, tb = _tc_head(pp, g3, dis16, b3.reshape(1, -1),
                      L1, lb1.reshape(1, -1), L2, lb2.reshape(1, -1),
                      M1, mb1.reshape(1, -1))

    ga, gb = _pairgather_sc(ta, tb, u_r, v_r)
    m2_pat = jnp.concatenate([M2[:, 0], jnp.zeros((8,), jnp.float32)])
    sel = jnp.kron(jnp.eye(128, dtype=jnp.float32), m2_pat.reshape(16, 1))
    y = _tc_final(ga.reshape(PPAD // PCH, PCH * 16),
                  gb.reshape(PPAD // PCH, PCH * 16),
                  sel, mb2.reshape(1, 1))
    return y.reshape(-1)[:P]


# speedup vs baseline: 1.5778x; 1.1772x over previous
import functools

import jax
import jax.numpy as jnp
from jax import lax
from jax.experimental import pallas as pl
from jax.experimental.pallas import tpu as pltpu
from jax.experimental.pallas import tpu_sc as plsc

N = 10000
E = 320000
P = 100000
NW = 32
EW = E // NW
CH = 128
NCH = 80
EPADW = NCH * CH
EPAD = NW * EPADW
NPAD = 10240
RT = NPAD // 16
ZB = 128
PCH = 128
PNCH = 26
PPAD = NW * PNCH * PCH

_mesh = plsc.VectorSubcoreMesh(core_axis_name="c", subcore_axis_name="s")


def _deg_sc(dst_r):
    @functools.partial(
        pl.kernel, mesh=_mesh,
        compiler_params=pltpu.CompilerParams(use_tc_tiling_on_sc=False),
        out_type=jax.ShapeDtypeStruct((2, NPAD, 128), jnp.float32),
        scratch_types=[
            pltpu.VMEM((NCH, CH), jnp.int32),
            pltpu.VMEM((CH, 16), jnp.float32),
            pltpu.VMEM((ZB, 16), jnp.float32),
            pltpu.VMEM_SHARED((NPAD, 16), jnp.float32),
        ],
    )
    def k(dstr_hbm, out_hbm, dstv, ones_v, zero_v, acc):
        c = lax.axis_index("c")
        s = lax.axis_index("s")
        wid = s * 2 + c
        pltpu.sync_copy(dstr_hbm.at[wid], dstv)

        def fill(i, _):
            ones_v[i] = jnp.full((16,), 1.0, jnp.float32)
            return 0
        lax.fori_loop(0, CH, fill, 0)

        def zfill(i, _):
            zero_v[i] = jnp.zeros((16,), jnp.float32)
            return 0
        lax.fori_loop(0, ZB, zfill, 0)
        for z in range(RT // ZB):
            pltpu.sync_copy(zero_v, acc.at[pl.ds(s * RT + z * ZB, ZB)])
        plsc.subcore_barrier()

        def chunk(j, _):
            pltpu.sync_copy(ones_v, acc.at[dstv.at[j]], add=True)
            return 0
        lax.fori_loop(0, NCH, chunk, 0)
        plsc.subcore_barrier()
        pltpu.sync_copy(acc.at[pl.ds(s * RT, RT)],
                        out_hbm.at[c, pl.ds(s * RT, RT), pl.ds(0, 16)])

    return k(dst_r)


def _scatter_sc(g, src_r, dst_r):
    D = g.shape[1]

    @functools.partial(
        pl.kernel, mesh=_mesh,
        compiler_params=pltpu.CompilerParams(use_tc_tiling_on_sc=False),
        out_type=jax.ShapeDtypeStruct((2, NPAD, 128), jnp.float32),
        scratch_types=[
            pltpu.VMEM((NCH, CH), jnp.int32),
            pltpu.VMEM((NCH, CH), jnp.int32),
            pltpu.VMEM((CH, D), jnp.float32),
            pltpu.VMEM((CH, D), jnp.float32),
            pltpu.VMEM((CH, D), jnp.float32),
            pltpu.VMEM((CH, D), jnp.float32),
            pltpu.VMEM((ZB, D), jnp.float32),
            pltpu.VMEM_SHARED((NPAD, D), jnp.float32),
            pltpu.SemaphoreType.DMA,
            pltpu.SemaphoreType.DMA,
            pltpu.SemaphoreType.DMA,
            pltpu.SemaphoreType.DMA,
        ],
    )
    def k(g_hbm, srcr_hbm, dstr_hbm, out_hbm, srcv, dstv, rows0, rows1,
          rows2, rows3, zero_v, acc, sem0, sem1, sem2, sem3):
        c = lax.axis_index("c")
        s = lax.axis_index("s")
        wid = s * 2 + c
        pltpu.sync_copy(srcr_hbm.at[wid], srcv)
        pltpu.sync_copy(dstr_hbm.at[wid], dstv)

        nsub = D // 16

        def zrow(t, _):
            zero_v[t // nsub, pl.ds((t % nsub) * 16, 16)] = jnp.zeros(
                (16,), jnp.float32)
            return 0
        lax.fori_loop(0, ZB * nsub, zrow, 0)
        for z in range(RT // ZB):
            pltpu.sync_copy(zero_v, acc.at[pl.ds(s * RT + z * ZB, ZB)])
        plsc.subcore_barrier()

        bufs = ((rows0, sem0), (rows1, sem1), (rows2, sem2), (rows3, sem3))
        pltpu.async_copy(g_hbm.at[srcv.at[0]], rows0, sem0)
        pltpu.async_copy(g_hbm.at[srcv.at[1]], rows1, sem1)

        def chunk4(jj, _):
            j0 = 4 * jj
            for b in range(4):
                rb, sb = bufs[b]
                rn, sn = bufs[(b + 2) % 4]
                pltpu.make_async_copy(g_hbm.at[srcv.at[j0 + b]], rb,
                                      sb).wait()

                @pl.when(j0 + b + 2 < NCH)
                def _():
                    pltpu.async_copy(g_hbm.at[srcv.at[j0 + b + 2]], rn, sn)

                pltpu.sync_copy(rb, acc.at[dstv.at[j0 + b]], add=True)
            return 0
        lax.fori_loop(0, NCH // 4, chunk4, 0)
        plsc.subcore_barrier()
        pltpu.sync_copy(acc.at[pl.ds(s * RT, RT)],
                        out_hbm.at[c, pl.ds(s * RT, RT), pl.ds(0, D)])

    return k(g, src_r, dst_r)


def _scatter2_sc(ga, gb, src_r2, dst_r2):
    NC2 = 2 * NCH

    @functools.partial(
        pl.kernel, mesh=_mesh,
        compiler_params=pltpu.CompilerParams(use_tc_tiling_on_sc=False),
        out_type=jax.ShapeDtypeStruct((2, NPAD, 128), jnp.float32),
        scratch_types=[
            pltpu.VMEM((NC2, CH), jnp.int32),
            pltpu.VMEM((NC2, CH), jnp.int32),
            pltpu.VMEM((CH, 64), jnp.float32),
            pltpu.VMEM((CH, 64), jnp.float32),
            pltpu.VMEM((CH, 64), jnp.float32),
            pltpu.VMEM((CH, 64), jnp.float32),
            pltpu.VMEM((ZB, 64), jnp.float32),
            pltpu.VMEM_SHARED((NPAD, 64), jnp.float32),
            pltpu.SemaphoreType.DMA,
            pltpu.SemaphoreType.DMA,
            pltpu.SemaphoreType.DMA,
            pltpu.SemaphoreType.DMA,
        ],
    )
    def k(ga_hbm, gb_hbm, srcr_hbm, dstr_hbm, out_hbm, srcv, dstv,
          rows0, rows1, rows2, rows3, zero_v, acc, sem0, sem1, sem2, sem3):
        c = lax.axis_index("c")
        s = lax.axis_index("s")
        pltpu.sync_copy(srcr_hbm.at[s], srcv)
        pltpu.sync_copy(dstr_hbm.at[s], dstv)

        def zrow(t, _):
            zero_v[t // 4, pl.ds((t % 4) * 16, 16)] = jnp.zeros(
                (16,), jnp.float32)
            return 0
        lax.fori_loop(0, ZB * 4, zrow, 0)
        for z in range(RT // ZB):
            pltpu.sync_copy(zero_v, acc.at[pl.ds(s * RT + z * ZB, ZB)])
        plsc.subcore_barrier()

        def run(tab):
            bufs = ((rows0, sem0), (rows1, sem1), (rows2, sem2),
                    (rows3, sem3))
            pltpu.async_copy(tab.at[srcv.at[0]], rows0, sem0)
            pltpu.async_copy(tab.at[srcv.at[1]], rows1, sem1)

            def chunk4(jj, _):
                j0 = 4 * jj
                for b in range(4):
                    rb, sb = bufs[b]
                    rn, sn = bufs[(b + 2) % 4]
                    pltpu.make_async_copy(tab.at[srcv.at[j0 + b]], rb,
                                          sb).wait()

                    @pl.when(j0 + b + 2 < NC2)
                    def _():
                        pltpu.async_copy(tab.at[srcv.at[j0 + b + 2]], rn, sn)

                    pltpu.sync_copy(rb, acc.at[dstv.at[j0 + b]], add=True)
                return 0
            lax.fori_loop(0, NC2 // 4, chunk4, 0)

        @pl.when(c == 0)
        def _():
            run(ga_hbm)

        @pl.when(c == 1)
        def _():
            run(gb_hbm)

        plsc.subcore_barrier()
        pltpu.sync_copy(acc.at[pl.ds(s * RT, RT)],
                        out_hbm.at[c, pl.ds(s * RT, RT), pl.ds(0, 64)])

    return k(ga, gb, src_r2, dst_r2)


def _pairgather_sc(ta, tb, u_r, v_r):
    @functools.partial(
        pl.kernel, mesh=_mesh,
        compiler_params=pltpu.CompilerParams(use_tc_tiling_on_sc=False),
        out_type=(jax.ShapeDtypeStruct((PPAD, 16), jnp.float32),
                  jax.ShapeDtypeStruct((PPAD, 16), jnp.float32)),
        scratch_types=[
            pltpu.VMEM((PNCH, PCH), jnp.int32),
            pltpu.VMEM((PNCH, PCH), jnp.int32),
            pltpu.VMEM((PCH, 16), jnp.float32),
            pltpu.VMEM((PCH, 16), jnp.float32),
            pltpu.VMEM((PCH, 16), jnp.float32),
            pltpu.VMEM((PCH, 16), jnp.float32),
            pltpu.SemaphoreType.DMA,
            pltpu.SemaphoreType.DMA,
            pltpu.SemaphoreType.DMA,
            pltpu.SemaphoreType.DMA,
        ],
    )
    def k(ta_hbm, tb_hbm, ur_hbm, vr_hbm, outa_hbm, outb_hbm, uv, vv,
          bufa0, bufb0, bufa1, bufb1, sa0, sb0, sa1, sb1):
        c = lax.axis_index("c")
        s = lax.axis_index("s")
        wid = s * 2 + c
        pltpu.sync_copy(ur_hbm.at[wid], uv)
        pltpu.sync_copy(vr_hbm.at[wid], vv)

        pltpu.async_copy(ta_hbm.at[uv.at[0]], bufa0, sa0)
        pltpu.async_copy(tb_hbm.at[vv.at[0]], bufb0, sb0)
        pltpu.async_copy(ta_hbm.at[uv.at[1]], bufa1, sa1)
        pltpu.async_copy(tb_hbm.at[vv.at[1]], bufb1, sb1)

        def chunk2(jj, _):
            j0 = 2 * jj
            for (j, ba, bb, sba, sbb) in ((j0, bufa0, bufb0, sa0, sb0),
                                          (j0 + 1, bufa1, bufb1, sa1, sb1)):
                base = (wid * PNCH + j) * PCH
                pltpu.make_async_copy(ta_hbm.at[uv.at[j]], ba, sba).wait()
                pltpu.make_async_copy(tb_hbm.at[vv.at[j]], bb, sbb).wait()
                pltpu.sync_copy(ba, outa_hbm.at[pl.ds(base, PCH)])
                pltpu.sync_copy(bb, outb_hbm.at[pl.ds(base, PCH)])

                @pl.when(j + 2 < PNCH)
                def _():
                    pltpu.async_copy(ta_hbm.at[uv.at[j + 2]], ba, sba)
                    pltpu.async_copy(tb_hbm.at[vv.at[j + 2]], bb, sbb)
            return 0
        lax.fori_loop(0, PNCH // 2, chunk2, 0)

    return k(ta, tb, u_r, v_r)


_BLK = 2000


def _tc_first(x, w1, degp):
    def body(x_ref, w_ref, d0_ref, d1_ref, ga_ref, gb_ref, dis_ref):
        deg = d0_ref[0, :, 0:1] + d1_ref[0, :, 0:1] + 1.0
        dis = lax.rsqrt(deg)
        h = jnp.dot(x_ref[...], w_ref[...], preferred_element_type=jnp.float32)
        g = dis * h
        ga_ref[...] = g[:, :64]
        gb_ref[...] = g[:, 64:]
        dis_ref[...] = jnp.broadcast_to(dis, (_BLK, 16))

    return pl.pallas_call(
        body,
        grid=(N // _BLK,),
        in_specs=[pl.BlockSpec((_BLK, 128), lambda i: (i, 0)),
                  pl.BlockSpec((128, 128), lambda i: (0, 0)),
                  pl.BlockSpec((1, _BLK, 128), lambda i: (0, i, 0)),
                  pl.BlockSpec((1, _BLK, 128), lambda i: (1, i, 0))],
        out_specs=[pl.BlockSpec((_BLK, 64), lambda i: (i, 0)),
                   pl.BlockSpec((_BLK, 64), lambda i: (i, 0)),
                   pl.BlockSpec((_BLK, 16), lambda i: (i, 0))],
        out_shape=[jax.ShapeDtypeStruct((N, 64), jnp.float32),
                   jax.ShapeDtypeStruct((N, 64), jnp.float32),
                   jax.ShapeDtypeStruct((N, 16), jnp.float32)],
    )(x, w1, degp, degp)


def _tc_mid2(pp, ga, gb, dis16, ba_row, bb_row, wa, wb):
    def body(pa_ref, pb_ref, ga_ref, gb_ref, dis_ref,
             ba_ref, bb_ref, wa_ref, wb_ref, out_ref):
        dis = dis_ref[:, 0:1]
        t_a = jnp.maximum(
            dis * (pa_ref[0, :, :64] + ga_ref[...]) + ba_ref[...], 0.0)
        t_b = jnp.maximum(
            dis * (pb_ref[0, :, :64] + gb_ref[...]) + bb_ref[...], 0.0)
        out_ref[...] = dis * (
            jnp.dot(t_a, wa_ref[...], preferred_element_type=jnp.float32)
            + jnp.dot(t_b, wb_ref[...], preferred_element_type=jnp.float32))

    blk64 = pl.BlockSpec((_BLK, 64), lambda i: (i, 0))
    pblk0 = pl.BlockSpec((1, _BLK, 128), lambda i: (0, i, 0))
    pblk1 = pl.BlockSpec((1, _BLK, 128), lambda i: (1, i, 0))
    return pl.pallas_call(
        body,
        grid=(N // _BLK,),
        in_specs=[pblk0, pblk1, blk64, blk64,
                  pl.BlockSpec((_BLK, 16), lambda i: (i, 0)),
                  pl.BlockSpec((1, 64), lambda i: (0, 0)),
                  pl.BlockSpec((1, 64), lambda i: (0, 0)),
                  pl.BlockSpec((64, 64), lambda i: (0, 0)),
                  pl.BlockSpec((64, 64), lambda i: (0, 0))],
        out_specs=pl.BlockSpec((_BLK, 64), lambda i: (i, 0)),
        out_shape=jax.ShapeDtypeStruct((N, 64), jnp.float32),
    )(pp, pp, ga, gb, dis16, ba_row, bb_row, wa, wb)


def _tc_mid(pp, g, dis16, b_row, w):
    din = g.shape[1]
    dout = w.shape[1]

    def body(p0_ref, p1_ref, g_ref, dis_ref, b_ref, w_ref, out_ref):
        dis = dis_ref[:, 0:1]
        p0 = p0_ref[0, :, :din]
        p1 = p1_ref[0, :, :din]
        t = dis * (p0 + p1 + g_ref[...]) + b_ref[...]
        t = jnp.maximum(t, 0.0)
        out_ref[...] = dis * jnp.dot(t, w_ref[...],
                                     preferred_element_type=jnp.float32)

    return pl.pallas_call(
        body,
        grid=(N // _BLK,),
        in_specs=[pl.BlockSpec((1, _BLK, 128), lambda i: (0, i, 0)),
                  pl.BlockSpec((1, _BLK, 128), lambda i: (1, i, 0)),
                  pl.BlockSpec((_BLK, din), lambda i: (i, 0)),
                  pl.BlockSpec((_BLK, 16), lambda i: (i, 0)),
                  pl.BlockSpec((1, din), lambda i: (0, 0)),
                  pl.BlockSpec((din, dout), lambda i: (0, 0))],
        out_specs=pl.BlockSpec((_BLK, dout), lambda i: (i, 0)),
        out_shape=jax.ShapeDtypeStruct((N, dout), jnp.float32),
    )(pp, pp, g, dis16, b_row, w)


def _tc_head(pp, g3, dis16, b3_row, l1, lb1_row, l2, lb2_row, m1, mb1_row):
    def body(p0_ref, p1_ref, g_ref, dis_ref, b3_ref, l1_ref, lb1_ref,
             l2_ref, lb2_ref, m1_ref, mb1_ref, ta_ref, tb_ref):
        dis = dis_ref[:, 0:1]
        o = dis * (p0_ref[0, :, :32] + p1_ref[0, :, :32] + g_ref[...]) \
            + b3_ref[...]
        o = jnp.maximum(o, 0.0)
        h4 = jnp.maximum(
            jnp.dot(o, l1_ref[...], preferred_element_type=jnp.float32)
            + lb1_ref[...], 0.0)
        emb = jnp.maximum(
            jnp.dot(h4, l2_ref[...], preferred_element_type=jnp.float32)
            + lb2_ref[...], 0.0)
        m1 = m1_ref[...]
        a = jnp.dot(emb, m1[:16, :], preferred_element_type=jnp.float32) \
            + mb1_ref[...]
        b = jnp.dot(emb, m1[16:, :], preferred_element_type=jnp.float32)
        ta_ref[...] = jnp.concatenate([a, a], axis=1)
        tb_ref[...] = jnp.concatenate([b, b], axis=1)

    return pl.pallas_call(
        body,
        grid=(N // _BLK,),
        in_specs=[pl.BlockSpec((1, _BLK, 128), lambda i: (0, i, 0)),
                  pl.BlockSpec((1, _BLK, 128), lambda i: (1, i, 0)),
                  pl.BlockSpec((_BLK, 32), lambda i: (i, 0)),
                  pl.BlockSpec((_BLK, 16), lambda i: (i, 0)),
                  pl.BlockSpec((1, 32), lambda i: (0, 0)),
                  pl.BlockSpec((32, 16), lambda i: (0, 0)),
                  pl.BlockSpec((1, 16), lambda i: (0, 0)),
                  pl.BlockSpec((16, 16), lambda i: (0, 0)),
                  pl.BlockSpec((1, 16), lambda i: (0, 0)),
                  pl.BlockSpec((32, 8), lambda i: (0, 0)),
                  pl.BlockSpec((1, 8), lambda i: (0, 0))],
        out_specs=[pl.BlockSpec((_BLK, 16), lambda i: (i, 0)),
                   pl.BlockSpec((_BLK, 16), lambda i: (i, 0))],
        out_shape=[jax.ShapeDtypeStruct((N, 16), jnp.float32),
                   jax.ShapeDtypeStruct((N, 16), jnp.float32)],
    )(pp, pp, g3, dis16, b3_row, l1, lb1_row, l2, lb2_row, m1, mb1_row)


def _tc_final(ga2, gb2, sel, mb2_s):
    rows = PPAD // 128
    blk = 104

    def body(a_ref, b_ref, sel_ref, mb2_ref, out_ref):
        t = jnp.maximum(a_ref[...] + b_ref[...], 0.0)
        sc = jnp.dot(t, sel_ref[...],
                     preferred_element_type=jnp.float32) + mb2_ref[...]
        out_ref[...] = 1.0 / (1.0 + jnp.exp(-sc))

    return pl.pallas_call(
        body,
        grid=(rows // blk,),
        in_specs=[pl.BlockSpec((blk, 2048), lambda i: (i, 0)),
                  pl.BlockSpec((blk, 2048), lambda i: (i, 0)),
                  pl.BlockSpec((2048, 128), lambda i: (0, 0)),
                  pl.BlockSpec((1, 1), lambda i: (0, 0))],
        out_specs=pl.BlockSpec((blk, 128), lambda i: (i, 0)),
        out_shape=jax.ShapeDtypeStruct((rows, 128), jnp.float32),
    )(ga2, gb2, sel, mb2_s)


def kernel(x, edge_index, pred_edges, W1, b1, W2, b2, W3, b3,
           L1, lb1, L2, lb2, M1, mb1, M2, mb2):
    ei = edge_index.astype(jnp.int32)
    npade = EPAD - E
    pad_src = jnp.arange(npade, dtype=jnp.int32) % N
    pad_dst = N + jnp.arange(npade, dtype=jnp.int32) % (NPAD - N)
    src_flat = jnp.concatenate([ei[0], pad_src])
    dst_flat = jnp.concatenate([ei[1], pad_dst])
    src_r = src_flat.reshape(NW, NCH, CH)
    dst_r = dst_flat.reshape(NW, NCH, CH)
    src_r2 = src_flat.reshape(16, 2 * NCH, CH)
    dst_r2 = dst_flat.reshape(16, 2 * NCH, CH)
    pe = pred_edges.astype(jnp.int32)
    npadp = PPAD - P
    pad_p = jnp.arange(npadp, dtype=jnp.int32) % N
    u_r = jnp.concatenate([pe[:, 0], pad_p]).reshape(NW, PNCH, PCH)
    v_r = jnp.concatenate([pe[:, 1], pad_p]).reshape(NW, PNCH, PCH)

    degp = _deg_sc(dst_r)
    g1a, g1b, dis16 = _tc_first(x, W1, degp)

    pp1 = _scatter2_sc(g1a, g1b, src_r2, dst_r2)
    g2 = _tc_mid2(pp1, g1a, g1b, dis16,
                  b1[:64].reshape(1, -1), b1[64:].reshape(1, -1),
                  W2[:64], W2[64:])

    pp = _scatter_sc(g2, src_r, dst_r)
    g3 = _tc_mid(pp, g2, dis16, b2.reshape(1, -1), W3)

    pp = _scatter_sc(g3, src_r, dst_r)
    ta, tb = _tc_head(pp, g3, dis16, b3.reshape(1, -1),
                      L1, lb1.reshape(1, -1), L2, lb2.reshape(1, -1),
                      M1, mb1.reshape(1, -1))

    ga, gb = _pairgather_sc(ta, tb, u_r, v_r)
    m2_pat = jnp.concatenate([M2[:, 0], jnp.zeros((8,), jnp.float32)])
    sel = jnp.kron(jnp.eye(128, dtype=jnp.float32), m2_pat.reshape(16, 1))
    y = _tc_final(ga.reshape(PPAD // PCH, PCH * 16),
                  gb.reshape(PPAD // PCH, PCH * 16),
                  sel, mb2.reshape(1, 1))
    return y.reshape(-1)[:P]
